# baseline (speedup 1.0000x reference)
import functools

import jax
import jax.numpy as jnp
from jax import lax
from jax.experimental import pallas as pl
from jax.experimental.pallas import tpu as pltpu
from jax.experimental.pallas import tpu_sc as plsc

_D = 28
_TOTAL = 4096 * 200
_NW = 32
_PER_W = _TOTAL // _NW
_CHUNK = 128
_NCHUNK = _PER_W // _CHUNK
_R8 = 8
_NSTREAM = 4


def _sc_gather(table8, idx2d):
    mesh = plsc.VectorSubcoreMesh(core_axis_name="c", subcore_axis_name="s")

    @functools.partial(
        pl.kernel,
        mesh=mesh,
        out_type=jax.ShapeDtypeStruct((_TOTAL, _NSTREAM * _R8), jnp.float32),
        scratch_types=[
            pltpu.VMEM((_CHUNK,), jnp.int32),
            pltpu.VMEM((_CHUNK,), jnp.int32),
            pltpu.VMEM((_CHUNK,), jnp.int32),
            pltpu.VMEM((_CHUNK,), jnp.int32),
            pltpu.VMEM((_CHUNK,), jnp.int32),
            pltpu.VMEM((_NSTREAM * _CHUNK, _R8), jnp.float32),
            pltpu.SemaphoreType.DMA,
        ],
        compiler_params=pltpu.CompilerParams(use_tc_tiling_on_sc=False),
    )
    def k(tbl_hbm, idx_hbm, out_hbm, idxc_v, gl0_v, gl1_v, gl2_v, gl3_v,
          land_v, sem):
        gls = (gl0_v, gl1_v, gl2_v, gl3_v)
        wid = lax.axis_index("s") * 2 + lax.axis_index("c")

        def body(c, carry):
            row = wid * _NCHUNK + c
            pltpu.sync_copy(idx_hbm.at[row], idxc_v)
            for m in range(_CHUNK // 16):
                v = idxc_v[pl.ds(m * 16, 16)]
                g = (7 * v) >> 1
                for kk in range(_NSTREAM):
                    gls[kk][pl.ds(m * 16, 16)] = g + kk
            cps = [
                pltpu.async_copy(
                    tbl_hbm.at[gls[kk]],
                    land_v.at[pl.ds(kk * _CHUNK, _CHUNK)],
                    sem,
                )
                for kk in range(_NSTREAM)
            ]
            for cp in cps:
                cp.wait()
            for kk in range(_NSTREAM):
                pltpu.sync_copy(
                    land_v.at[pl.ds(kk * _CHUNK, _CHUNK)],
                    out_hbm.at[pl.ds(row * _CHUNK, _CHUNK), pl.ds(kk * _R8, _R8)],
                )
            return carry

        lax.fori_loop(0, _NCHUNK, body, 0)

    return k(table8, idx2d)


def kernel(table, indices):
    idx = indices.astype(jnp.int32).reshape(_TOTAL // _CHUNK, _CHUNK)
    table8 = table.reshape(-1, _R8)
    land = _sc_gather(table8, idx)
    parity = (idx.reshape(_TOTAL, 1) & 1) == 1
    out = jnp.where(parity, land[:, 4:4 + _D], land[:, :_D])
    return out.reshape(indices.shape[0], indices.shape[1], _D)


# speedup vs baseline: 1.1545x; 1.1545x over previous
import functools

import jax
import jax.numpy as jnp
from jax import lax
from jax.experimental import pallas as pl
from jax.experimental.pallas import tpu as pltpu
from jax.experimental.pallas import tpu_sc as plsc

_D = 28
_DP = 32
_TOTAL = 4096 * 200
_NW = 32
_PER_W = _TOTAL // _NW
_CHUNK = 128
_NCHUNK = _PER_W // _CHUNK
_NBUF = 4
_LEAD = 2


def _sc_gather(table_pad, idx2d):
    mesh = plsc.VectorSubcoreMesh(core_axis_name="c", subcore_axis_name="s")

    @functools.partial(
        pl.kernel,
        mesh=mesh,
        out_type=jax.ShapeDtypeStruct((_TOTAL, _DP), jnp.float32),
        scratch_types=[
            pltpu.VMEM((_NCHUNK, _CHUNK), jnp.int32),
            pltpu.VMEM((_NBUF, _CHUNK, _DP), jnp.float32),
            pltpu.SemaphoreType.DMA,
            pltpu.SemaphoreType.DMA,
            pltpu.SemaphoreType.DMA,
            pltpu.SemaphoreType.DMA,
            pltpu.SemaphoreType.DMA,
            pltpu.SemaphoreType.DMA,
            pltpu.SemaphoreType.DMA,
            pltpu.SemaphoreType.DMA,
        ],
        compiler_params=pltpu.CompilerParams(use_tc_tiling_on_sc=False),
    )
    def k(tbl_hbm, idx_hbm, out_hbm, idx_v, land_v, g0, g1, g2, g3,
          w0, w1, w2, w3):
        gsem = (g0, g1, g2, g3)
        wsem = (w0, w1, w2, w3)
        wid = lax.axis_index("s") * 2 + lax.axis_index("c")
        base = wid * _PER_W
        pltpu.sync_copy(idx_hbm.at[pl.ds(wid * _NCHUNK, _NCHUNK)], idx_v)

        def fire_gather(b, c):
            pltpu.async_copy(tbl_hbm.at[idx_v.at[c]], land_v.at[b], gsem[b])

        def fire_write(b, c):
            pltpu.async_copy(
                land_v.at[b],
                out_hbm.at[pl.ds(base + c * _CHUNK, _CHUNK)],
                wsem[b],
            )

        def wait_gather(b):
            pltpu.make_async_copy(
                out_hbm.at[pl.ds(0, _CHUNK)], land_v.at[b], gsem[b]
            ).wait()

        def wait_write(b):
            pltpu.make_async_copy(
                land_v.at[b], out_hbm.at[pl.ds(0, _CHUNK)], wsem[b]
            ).wait()

        for c0 in range(_LEAD):
            fire_gather(c0 % _NBUF, c0)

        def body(c4, carry):
            for boff in range(_NBUF):
                g = c4 * _NBUF + boff
                blead = (boff + _LEAD) % _NBUF

                @pl.when(g + _LEAD < _NCHUNK)
                def _():
                    @pl.when(g + _LEAD >= _NBUF)
                    def _():
                        wait_write(blead)
                    fire_gather(blead, g + _LEAD)

                wait_gather(boff)
                fire_write(boff, g)
            return carry

        lax.fori_loop(0, _NCHUNK // _NBUF, body, 0)
        for b in range(_NBUF):
            wait_write(b)

    return k(table_pad, idx2d)


def _tc_compact(land):
    blk = 4096

    def body(in_ref, out_ref):
        out_ref[...] = in_ref[:, :_D]

    return pl.pallas_call(
        body,
        grid=(_TOTAL // blk,),
        in_specs=[pl.BlockSpec((blk, _DP), lambda i: (i, 0))],
        out_specs=pl.BlockSpec((blk, _D), lambda i: (i, 0)),
        out_shape=jax.ShapeDtypeStruct((_TOTAL, _D), jnp.float32),
    )(land)


def kernel(table, indices):
    idx = indices.astype(jnp.int32).reshape(_TOTAL // _CHUNK, _CHUNK)
    table_pad = jnp.pad(table, ((0, 0), (0, _DP - _D)))
    land = _sc_gather(table_pad, idx)
    out = _tc_compact(land)
    return out.reshape(indices.shape[0], indices.shape[1], _D)


# speedup vs baseline: 1.1650x; 1.0091x over previous
import functools

import jax
import jax.numpy as jnp
from jax import lax
from jax.experimental import pallas as pl
from jax.experimental.pallas import tpu as pltpu
from jax.experimental.pallas import tpu_sc as plsc

_D = 28
_DP = 32
_TOTAL = 4096 * 200
_NW = 32
_PER_W = _TOTAL // _NW
_CHUNK = 128
_NCHUNK = _PER_W // _CHUNK
_R8 = 8
_NS = 4
_NBUF = 4


def _sc_gather(table8, idx2d):
    mesh = plsc.VectorSubcoreMesh(core_axis_name="c", subcore_axis_name="s")

    scratch = (
        [pltpu.VMEM((_CHUNK,), jnp.int32) for _ in range(_NBUF)]
        + [pltpu.VMEM((_CHUNK,), jnp.int32) for _ in range(_NBUF * _NS)]
        + [pltpu.VMEM((_NBUF, _NS * _CHUNK, _R8), jnp.float32)]
        + [pltpu.SemaphoreType.DMA] * (3 * _NBUF)
    )

    @functools.partial(
        pl.kernel,
        mesh=mesh,
        out_type=jax.ShapeDtypeStruct((_TOTAL, _DP), jnp.float32),
        scratch_types=scratch,
        compiler_params=pltpu.CompilerParams(use_tc_tiling_on_sc=False),
    )
    def k(tbl_hbm, idx_hbm, out_hbm, *refs):
        idxc = refs[0:_NBUF]
        gl = refs[_NBUF:_NBUF + _NBUF * _NS]
        land_v = refs[_NBUF + _NBUF * _NS]
        isem = refs[_NBUF + _NBUF * _NS + 1:][0:_NBUF]
        gsem = refs[_NBUF + _NBUF * _NS + 1:][_NBUF:2 * _NBUF]
        wsem = refs[_NBUF + _NBUF * _NS + 1:][2 * _NBUF:3 * _NBUF]
        wid = lax.axis_index("s") * 2 + lax.axis_index("c")
        base = wid * _NCHUNK

        def fire_idx(b, c):
            pltpu.async_copy(idx_hbm.at[base + c], idxc[b], isem[b])

        def wait_idx(b):
            pltpu.make_async_copy(idx_hbm.at[0], idxc[b], isem[b]).wait()

        def build_and_fire(b, c):
            for m in range(_CHUNK // 16):
                v = idxc[b][pl.ds(m * 16, 16)]
                g = (7 * v) >> 1
                for kk in range(_NS):
                    gl[b * _NS + kk][pl.ds(m * 16, 16)] = g + kk
            for kk in range(_NS):
                pltpu.async_copy(
                    tbl_hbm.at[gl[b * _NS + kk]],
                    land_v.at[b, pl.ds(kk * _CHUNK, _CHUNK)],
                    gsem[b],
                )

        def wait_gathers(b):
            pltpu.make_async_copy(
                tbl_hbm.at[pl.ds(0, _NS * _CHUNK)], land_v.at[b], gsem[b]
            ).wait()

        def fire_writes(b, c):
            for kk in range(_NS):
                pltpu.async_copy(
                    land_v.at[b, pl.ds(kk * _CHUNK, _CHUNK)],
                    out_hbm.at[pl.ds((base + c) * _CHUNK, _CHUNK),
                               pl.ds(kk * _R8, _R8)],
                    wsem[b],
                )

        def wait_writes(b):
            pltpu.make_async_copy(
                land_v.at[b], tbl_hbm.at[pl.ds(0, _NS * _CHUNK)], wsem[b]
            ).wait()

        for c0 in range(3):
            fire_idx(c0, c0)
        for c0 in range(2):
            wait_idx(c0)
            build_and_fire(c0, c0)

        def body(g4, carry):
            for boff in range(_NBUF):
                g = g4 * _NBUF + boff
                b3 = (boff + 3) % _NBUF
                b2 = (boff + 2) % _NBUF

                @pl.when(g + 3 < _NCHUNK)
                def _():
                    fire_idx(b3, g + 3)

                @pl.when(g + 2 < _NCHUNK)
                def _():
                    wait_idx(b2)

                    @pl.when(g + 2 >= _NBUF)
                    def _():
                        wait_writes(b2)

                    build_and_fire(b2, g + 2)

                wait_gathers(boff)
                fire_writes(boff, g)
            return carry

        lax.fori_loop(0, _NCHUNK // _NBUF, body, 0)
        for b in range(_NBUF):
            wait_writes(b)

    return k(table8, idx2d)


def kernel(table, indices):
    idx = indices.astype(jnp.int32).reshape(_TOTAL // _CHUNK, _CHUNK)
    table8 = table.reshape(-1, _R8)
    land = _sc_gather(table8, idx)
    parity = (idx.reshape(_TOTAL, 1) & 1) == 1
    out = jnp.where(parity, land[:, 4:4 + _D], land[:, :_D])
    return out.reshape(indices.shape[0], indices.shape[1], _D)


# speedup vs baseline: 1.3154x; 1.1291x over previous
import functools

import jax
import jax.numpy as jnp
from jax import lax
from jax.experimental import pallas as pl
from jax.experimental.pallas import tpu as pltpu
from jax.experimental.pallas import tpu_sc as plsc

_D = 28
_DP = 32
_TOTAL = 4096 * 200
_NW = 32
_PER_W = _TOTAL // _NW
_CHUNK = 128
_NCHUNK = _PER_W // _CHUNK
_NBUF = 4
_LEAD = 2


def _sc_gather(table_pad, idx2d):
    mesh = plsc.VectorSubcoreMesh(core_axis_name="c", subcore_axis_name="s")

    @functools.partial(
        pl.kernel,
        mesh=mesh,
        out_type=jax.ShapeDtypeStruct((_TOTAL, _DP), jnp.float32),
        scratch_types=[
            pltpu.VMEM((_NCHUNK, _CHUNK), jnp.int32),
            pltpu.VMEM((_NBUF, _CHUNK, _DP), jnp.float32),
            pltpu.SemaphoreType.DMA,
            pltpu.SemaphoreType.DMA,
            pltpu.SemaphoreType.DMA,
            pltpu.SemaphoreType.DMA,
            pltpu.SemaphoreType.DMA,
            pltpu.SemaphoreType.DMA,
            pltpu.SemaphoreType.DMA,
            pltpu.SemaphoreType.DMA,
        ],
        compiler_params=pltpu.CompilerParams(use_tc_tiling_on_sc=False),
    )
    def k(tbl_hbm, idx_hbm, out_hbm, idx_v, land_v, g0, g1, g2, g3,
          w0, w1, w2, w3):
        gsem = (g0, g1, g2, g3)
        wsem = (w0, w1, w2, w3)
        wid = lax.axis_index("s") * 2 + lax.axis_index("c")
        base = wid * _PER_W
        pltpu.sync_copy(idx_hbm.at[pl.ds(wid * _NCHUNK, _NCHUNK)], idx_v)

        def fire_gather(b, c):
            pltpu.async_copy(tbl_hbm.at[idx_v.at[c]], land_v.at[b], gsem[b])

        def fire_write(b, c):
            pltpu.async_copy(
                land_v.at[b],
                out_hbm.at[pl.ds(base + c * _CHUNK, _CHUNK)],
                wsem[b],
            )

        def wait_gather(b):
            pltpu.make_async_copy(
                out_hbm.at[pl.ds(0, _CHUNK)], land_v.at[b], gsem[b]
            ).wait()

        def wait_write(b):
            pltpu.make_async_copy(
                land_v.at[b], out_hbm.at[pl.ds(0, _CHUNK)], wsem[b]
            ).wait()

        for c0 in range(_LEAD):
            fire_gather(c0 % _NBUF, c0)

        def body(c4, carry):
            for boff in range(_NBUF):
                g = c4 * _NBUF + boff
                blead = (boff + _LEAD) % _NBUF

                @pl.when(g + _LEAD < _NCHUNK)
                def _():
                    @pl.when(g + _LEAD >= _NBUF)
                    def _():
                        wait_write(blead)
                    fire_gather(blead, g + _LEAD)

                wait_gather(boff)
                fire_write(boff, g)
            return carry

        lax.fori_loop(0, _NCHUNK // _NBUF, body, 0)
        for b in range(_NBUF):
            wait_write(b)

    return k(table_pad, idx2d)


def kernel(table, indices):
    idx = indices.astype(jnp.int32).reshape(_TOTAL // _CHUNK, _CHUNK)
    table_pad = jnp.pad(table, ((0, 0), (0, _DP - _D)))
    land = _sc_gather(table_pad, idx)
    return land[:, :_D].reshape(indices.shape[0], indices.shape[1], _D)
